# baseline (speedup 1.0000x reference)
import functools

import jax
import jax.numpy as jnp
from jax import lax
from jax.experimental import pallas as pl
from jax.experimental.pallas import tpu as pltpu
from jax.experimental.pallas import tpu_sc as plsc

_VOCAB = 1_000_000
_D = 64
_B = 16384
_W = 128
_BROWS = _VOCAB // 2
_BBIAS = (_VOCAB + _W - 1) // _W
_NC = 1
_NS = 16
_L = 16
_NW = _NC * _NS
_BPW = _B // _NW
_Q = 128
_NQ = _BPW // _Q
_NGRP = _Q // _L

_mesh = plsc.VectorSubcoreMesh(
    core_axis_name="c", subcore_axis_name="s", num_cores=_NC, num_subcores=_NS
)


@functools.partial(
    pl.kernel,
    out_type=jax.ShapeDtypeStruct((_B,), jnp.float32),
    mesh=_mesh,
    compiler_params=pltpu.CompilerParams(needs_layout_passes=False),
    scratch_types=[
        pltpu.VMEM((_BPW,), jnp.int32),
        pltpu.VMEM((_BPW,), jnp.int32),
        pltpu.VMEM((_BPW,), jnp.int32),
        pltpu.VMEM((_BPW,), jnp.int32),
        pltpu.VMEM((_BPW,), jnp.int32),
        pltpu.VMEM((_BPW,), jnp.int32),
        pltpu.VMEM((_Q, _W), jnp.float32),
        pltpu.VMEM((_Q, _W), jnp.float32),
        pltpu.VMEM((_Q, _W), jnp.float32),
        pltpu.VMEM((_Q, _W), jnp.float32),
        pltpu.VMEM((_BPW,), jnp.float32),
        pltpu.SemaphoreType.DMA,
        pltpu.SemaphoreType.DMA,
        pltpu.SemaphoreType.DMA,
        pltpu.SemaphoreType.DMA,
    ],
)
def _glove_sc(left_hbm, right_hbm, ltab_hbm, lbias_hbm, rtab_hbm, rbias_hbm,
              out_hbm, lids, rids, ltix, rtix, lbix, rbix,
              lrow, rrow, lbrow, rbrow, outv, sem0, sem1, sem2, sem3):
    wid = lax.axis_index("s") * _NC + lax.axis_index("c")
    base = wid * _BPW

    pltpu.sync_copy(left_hbm.at[pl.ds(base, _BPW)], lids)
    pltpu.sync_copy(right_hbm.at[pl.ds(base, _BPW)], rids)

    def derive(g, carry):
        gb = pl.multiple_of(g * _L, _L)
        vl = lids[pl.ds(gb, _L)]
        vr = rids[pl.ds(gb, _L)]
        ltix[pl.ds(gb, _L)] = vl >> 1
        rtix[pl.ds(gb, _L)] = vr >> 1
        lbix[pl.ds(gb, _L)] = vl >> 7
        rbix[pl.ds(gb, _L)] = vr >> 7
        return carry

    lax.fori_loop(0, _BPW // _L, derive, 0)

    lane = lax.iota(jnp.int32, _L)

    for q in range(_NQ):
        qb = q * _Q
        c0 = pltpu.async_copy(ltab_hbm.at[ltix.at[pl.ds(qb, _Q)]], lrow, sem0)
        c1 = pltpu.async_copy(rtab_hbm.at[rtix.at[pl.ds(qb, _Q)]], rrow, sem1)
        c2 = pltpu.async_copy(lbias_hbm.at[lbix.at[pl.ds(qb, _Q)]], lbrow, sem2)
        c3 = pltpu.async_copy(rbias_hbm.at[rbix.at[pl.ds(qb, _Q)]], rbrow, sem3)
        c0.wait()
        c1.wait()
        c2.wait()
        c3.wait()

        def group(g, carry):
            gb = pl.multiple_of(g * _L, _L)
            rows = jnp.full((_L,), g * _L, jnp.int32) + lane
            vl = lids[pl.ds(pl.multiple_of(qb + gb, _L), _L)]
            vr = rids[pl.ds(pl.multiple_of(qb + gb, _L), _L)]
            lhalf = (vl & 1) * _D
            rhalf = (vr & 1) * _D
            acc = plsc.load_gather(lbrow, [rows, vl & (_W - 1)]) + plsc.load_gather(
                rbrow, [rows, vr & (_W - 1)])
            for c in range(_D):
                acc = acc + plsc.load_gather(lrow, [rows, lhalf + c]) * plsc.load_gather(
                    rrow, [rows, rhalf + c])
            outv[pl.ds(pl.multiple_of(qb + gb, _L), _L)] = acc
            return carry

        lax.fori_loop(0, _NGRP, group, 0)

    pltpu.sync_copy(outv, out_hbm.at[pl.ds(base, _BPW)])


def kernel(left_id, right_id, l_emb, l_bias, r_emb, r_bias):
    ltab = l_emb.reshape(_BROWS, _W)
    rtab = r_emb.reshape(_BROWS, _W)
    pad = _BBIAS * _W - _VOCAB
    lbias2 = jnp.pad(l_bias.reshape(_VOCAB), (0, pad)).reshape(_BBIAS, _W)
    rbias2 = jnp.pad(r_bias.reshape(_VOCAB), (0, pad)).reshape(_BBIAS, _W)
    return _glove_sc(
        left_id.astype(jnp.int32), right_id.astype(jnp.int32),
        ltab, lbias2, rtab, rbias2,
    )


# speedup vs baseline: 1.0234x; 1.0234x over previous
import functools

import jax
import jax.numpy as jnp
from jax import lax
from jax.experimental import pallas as pl
from jax.experimental.pallas import tpu as pltpu
from jax.experimental.pallas import tpu_sc as plsc

_VOCAB = 1_000_000
_D = 64
_B = 16384
_W = 128
_BROWS = _VOCAB // 2
_BBIAS = (_VOCAB + _W - 1) // _W
_NC = 2
_NS = 16
_L = 16
_NW = _NC * _NS
_BPW = _B // _NW
_Q = 128
_NQ = _BPW // _Q
_NGRP = _Q // _L

_mesh = plsc.VectorSubcoreMesh(
    core_axis_name="c", subcore_axis_name="s", num_cores=_NC, num_subcores=_NS
)
_params = pltpu.CompilerParams(needs_layout_passes=False)


@functools.partial(
    pl.kernel,
    out_type=(
        jax.ShapeDtypeStruct((_D, _B), jnp.float32),
        jax.ShapeDtypeStruct((_B,), jnp.float32),
    ),
    mesh=_mesh,
    compiler_params=_params,
    scratch_types=[
        pltpu.VMEM((_BPW,), jnp.int32),
        pltpu.VMEM((_BPW,), jnp.int32),
        pltpu.VMEM((_BPW,), jnp.int32),
        pltpu.VMEM((_Q, _W), jnp.float32),
        pltpu.VMEM((_Q, _W), jnp.float32),
        pltpu.VMEM((_D, _BPW), jnp.float32),
        pltpu.VMEM((_BPW,), jnp.float32),
        pltpu.SemaphoreType.DMA,
        pltpu.SemaphoreType.DMA,
    ],
)
def _gather_sc(id_hbm, tab_hbm, bias_hbm, stage_hbm, bvec_hbm,
               ids, tix, bix, qrow, bqrow, vstage, bv, sem0, sem1):
    wid = lax.axis_index("s") * _NC + lax.axis_index("c")
    base = wid * _BPW

    pltpu.sync_copy(id_hbm.at[pl.ds(base, _BPW)], ids)

    def derive(g, carry):
        gb = pl.multiple_of(g * _L, _L)
        v = ids[pl.ds(gb, _L)]
        tix[pl.ds(gb, _L)] = v >> 1
        bix[pl.ds(gb, _L)] = v >> 7
        return carry

    lax.fori_loop(0, _BPW // _L, derive, 0)

    lane = lax.iota(jnp.int32, _L)

    for q in range(_NQ):
        qb = q * _Q
        c0 = pltpu.async_copy(tab_hbm.at[tix.at[pl.ds(qb, _Q)]], qrow, sem0)
        c1 = pltpu.async_copy(bias_hbm.at[bix.at[pl.ds(qb, _Q)]], bqrow, sem1)
        c0.wait()
        c1.wait()

        def group(g, carry):
            gb = pl.multiple_of(g * _L, _L)
            rows = jnp.full((_L,), g * _L, jnp.int32) + lane
            v = ids[pl.ds(pl.multiple_of(qb + gb, _L), _L)]
            half = (v & 1) * _D
            bv[pl.ds(pl.multiple_of(qb + gb, _L), _L)] = plsc.load_gather(
                bqrow, [rows, v & (_W - 1)])
            for c in range(_D):
                vstage[c, pl.ds(pl.multiple_of(qb + gb, _L), _L)] = (
                    plsc.load_gather(qrow, [rows, half + c]))
            return carry

        lax.fori_loop(0, _NGRP, group, 0)

    pltpu.sync_copy(vstage, stage_hbm.at[:, pl.ds(base, _BPW)])
    pltpu.sync_copy(bv, bvec_hbm.at[pl.ds(base, _BPW)])


@functools.partial(
    pl.kernel,
    out_type=jax.ShapeDtypeStruct((_B,), jnp.float32),
    mesh=_mesh,
    compiler_params=_params,
    scratch_types=[
        pltpu.VMEM((_BPW,), jnp.int32),
        pltpu.VMEM((_BPW,), jnp.int32),
        pltpu.VMEM((_BPW,), jnp.int32),
        pltpu.VMEM((_Q, _W), jnp.float32),
        pltpu.VMEM((_Q, _W), jnp.float32),
        pltpu.VMEM((_D, _BPW), jnp.float32),
        pltpu.VMEM((_BPW,), jnp.float32),
        pltpu.VMEM((_BPW,), jnp.float32),
        pltpu.SemaphoreType.DMA,
        pltpu.SemaphoreType.DMA,
        pltpu.SemaphoreType.DMA,
    ],
)
def _dot_sc(id_hbm, tab_hbm, bias_hbm, stage_hbm, bvec_hbm, out_hbm,
            ids, tix, bix, qrow, bqrow, lsv, lbv, outv, sem0, sem1, sem2):
    wid = lax.axis_index("s") * _NC + lax.axis_index("c")
    base = wid * _BPW

    pltpu.sync_copy(id_hbm.at[pl.ds(base, _BPW)], ids)
    cs = pltpu.async_copy(stage_hbm.at[:, pl.ds(base, _BPW)], lsv, sem2)
    pltpu.sync_copy(bvec_hbm.at[pl.ds(base, _BPW)], lbv)

    def derive(g, carry):
        gb = pl.multiple_of(g * _L, _L)
        v = ids[pl.ds(gb, _L)]
        tix[pl.ds(gb, _L)] = v >> 1
        bix[pl.ds(gb, _L)] = v >> 7
        return carry

    lax.fori_loop(0, _BPW // _L, derive, 0)
    cs.wait()

    lane = lax.iota(jnp.int32, _L)

    for q in range(_NQ):
        qb = q * _Q
        c0 = pltpu.async_copy(tab_hbm.at[tix.at[pl.ds(qb, _Q)]], qrow, sem0)
        c1 = pltpu.async_copy(bias_hbm.at[bix.at[pl.ds(qb, _Q)]], bqrow, sem1)
        c0.wait()
        c1.wait()

        def group(g, carry):
            gb = pl.multiple_of(g * _L, _L)
            ab = pl.multiple_of(qb + gb, _L)
            rows = jnp.full((_L,), g * _L, jnp.int32) + lane
            v = ids[pl.ds(ab, _L)]
            half = (v & 1) * _D
            acc = lbv[pl.ds(ab, _L)] + plsc.load_gather(
                bqrow, [rows, v & (_W - 1)])
            for c in range(_D):
                acc = acc + plsc.load_gather(qrow, [rows, half + c]) * lsv[
                    c, pl.ds(ab, _L)]
            outv[pl.ds(ab, _L)] = acc
            return carry

        lax.fori_loop(0, _NGRP, group, 0)

    pltpu.sync_copy(outv, out_hbm.at[pl.ds(base, _BPW)])


def kernel(left_id, right_id, l_emb, l_bias, r_emb, r_bias):
    ltab = l_emb.reshape(_BROWS, _W)
    rtab = r_emb.reshape(_BROWS, _W)
    pad = _BBIAS * _W - _VOCAB
    lbias2 = jnp.pad(l_bias.reshape(_VOCAB), (0, pad)).reshape(_BBIAS, _W)
    rbias2 = jnp.pad(r_bias.reshape(_VOCAB), (0, pad)).reshape(_BBIAS, _W)
    lstage, lbvec = _gather_sc(left_id.astype(jnp.int32), ltab, lbias2)
    return _dot_sc(
        right_id.astype(jnp.int32), rtab, rbias2, lstage, lbvec
    )


# speedup vs baseline: 2.6245x; 2.5643x over previous
import functools

import jax
import jax.numpy as jnp
from jax import lax
from jax.experimental import pallas as pl
from jax.experimental.pallas import tpu as pltpu
from jax.experimental.pallas import tpu_sc as plsc

_VOCAB = 1_000_000
_D = 64
_B = 16384
_W = 128
_NC = 2
_NS = 16
_L = 16
_NW = _NC * _NS
_BPW = _B // _NW
_NG = _BPW // _L
_R = 4

_mesh = plsc.VectorSubcoreMesh(
    core_axis_name="c", subcore_axis_name="s", num_cores=_NC, num_subcores=_NS
)


@functools.partial(
    pl.kernel,
    out_type=jax.ShapeDtypeStruct((_B,), jnp.float32),
    mesh=_mesh,
    compiler_params=pltpu.CompilerParams(needs_layout_passes=False),
    scratch_types=[
        pltpu.VMEM((_BPW,), jnp.int32),
        pltpu.VMEM((_BPW,), jnp.int32),
        pltpu.VMEM((_R, _D, _W), jnp.float32),
        pltpu.VMEM((_R, _D, _W), jnp.float32),
        pltpu.VMEM((_L, _W), jnp.float32),
        pltpu.VMEM((_L, _W), jnp.float32),
        pltpu.VMEM((_L, _D), jnp.float32),
        pltpu.VMEM((_L, _D), jnp.float32),
        pltpu.VMEM((_BPW,), jnp.float32),
        [pltpu.SemaphoreType.DMA] * _R,
        [pltpu.SemaphoreType.DMA] * _R,
        pltpu.SemaphoreType.DMA,
        pltpu.SemaphoreType.DMA,
    ],
)
def _glove_sc(left_hbm, right_hbm, ltab_hbm, lbias_hbm, rtab_hbm, rbias_hbm,
              out_hbm, lids, rids, lblk, rblk, lbst, rbst, lstage, rstage,
              outv, lsems, rsems, lbsem, rbsem):
    wid = lax.axis_index("s") * _NC + lax.axis_index("c")
    base = wid * _BPW

    pltpu.sync_copy(left_hbm.at[pl.ds(base, _BPW)], lids)
    pltpu.sync_copy(right_hbm.at[pl.ds(base, _BPW)], rids)

    lane = lax.iota(jnp.int32, _L)
    cvec = lax.iota(jnp.int32, _L)

    def group(g, carry):
        gb = pl.multiple_of(g * _L, _L)
        vl = lids[pl.ds(gb, _L)]
        vr = rids[pl.ds(gb, _L)]
        lblks = (vl >> 7) << 7
        rblks = (vr >> 7) << 7
        llanes = vl & (_W - 1)
        rlanes = vr & (_W - 1)

        def fire(j):
            slot = j % _R
            cl = pltpu.async_copy(
                ltab_hbm.at[:, pl.ds(pl.multiple_of(lblks[j], _W), _W)],
                lblk.at[slot], lsems[slot])
            cr = pltpu.async_copy(
                rtab_hbm.at[:, pl.ds(pl.multiple_of(rblks[j], _W), _W)],
                rblk.at[slot], rsems[slot])
            return cl, cr

        bias_copies = []
        for j in range(_L):
            bias_copies.append(pltpu.async_copy(
                lbias_hbm.at[:, pl.ds(pl.multiple_of(lblks[j], _W), _W)],
                lbst.at[pl.ds(j, 1)], lbsem))
            bias_copies.append(pltpu.async_copy(
                rbias_hbm.at[:, pl.ds(pl.multiple_of(rblks[j], _W), _W)],
                rbst.at[pl.ds(j, 1)], rbsem))

        inflight = [fire(j) for j in range(_R)]
        for j in range(_L):
            slot = j % _R
            cl, cr = inflight[j]
            cl.wait()
            cr.wait()
            lj = llanes[j]
            rj = rlanes[j]
            for cc in range(0, _D, _L):
                lstage[j, pl.ds(cc, _L)] = plsc.load_gather(
                    lblk.at[slot], [cvec + cc, jnp.full((_L,), 0, jnp.int32) + lj])
                rstage[j, pl.ds(cc, _L)] = plsc.load_gather(
                    rblk.at[slot], [cvec + cc, jnp.full((_L,), 0, jnp.int32) + rj])
            if j + _R < _L:
                inflight.append(fire(j + _R))

        for c in bias_copies:
            c.wait()

        acc = plsc.load_gather(lbst, [lane, llanes]) + plsc.load_gather(
            rbst, [lane, rlanes])
        for c in range(_D):
            col = jnp.full((_L,), c, jnp.int32)
            acc = acc + plsc.load_gather(lstage, [lane, col]) * plsc.load_gather(
                rstage, [lane, col])
        outv[pl.ds(gb, _L)] = acc
        return carry

    lax.fori_loop(0, _NG, group, 0)

    pltpu.sync_copy(outv, out_hbm.at[pl.ds(base, _BPW)])


def kernel(left_id, right_id, l_emb, l_bias, r_emb, r_bias):
    return _glove_sc(
        left_id.astype(jnp.int32), right_id.astype(jnp.int32),
        l_emb.T, l_bias.T, r_emb.T, r_bias.T,
    )


# speedup vs baseline: 2.6355x; 1.0042x over previous
import functools

import jax
import jax.numpy as jnp
from jax import lax
from jax.experimental import pallas as pl
from jax.experimental.pallas import tpu as pltpu
from jax.experimental.pallas import tpu_sc as plsc

_VOCAB = 1_000_000
_D = 64
_B = 16384
_W = 128
_NC = 2
_NS = 16
_L = 16
_NW = _NC * _NS
_BPW = _B // _NW
_NG = _BPW // _L
_R = 6

_mesh = plsc.VectorSubcoreMesh(
    core_axis_name="c", subcore_axis_name="s", num_cores=_NC, num_subcores=_NS
)


@functools.partial(
    pl.kernel,
    out_type=jax.ShapeDtypeStruct((_B,), jnp.float32),
    mesh=_mesh,
    compiler_params=pltpu.CompilerParams(needs_layout_passes=False),
    scratch_types=[
        pltpu.VMEM((_BPW,), jnp.int32),
        pltpu.VMEM((_BPW,), jnp.int32),
        pltpu.VMEM((_R, _D, _W), jnp.float32),
        pltpu.VMEM((_R, _D, _W), jnp.float32),
        pltpu.VMEM((_L, _W), jnp.float32),
        pltpu.VMEM((_L, _W), jnp.float32),
        pltpu.VMEM((_L, _D), jnp.float32),
        pltpu.VMEM((_L, _D), jnp.float32),
        pltpu.VMEM((_BPW,), jnp.float32),
        [pltpu.SemaphoreType.DMA] * _R,
        [pltpu.SemaphoreType.DMA] * _R,
        pltpu.SemaphoreType.DMA,
        pltpu.SemaphoreType.DMA,
    ],
)
def _glove_sc(left_hbm, right_hbm, ltab_hbm, lbias_hbm, rtab_hbm, rbias_hbm,
              out_hbm, lids, rids, lblk, rblk, lbst, rbst, lstage, rstage,
              outv, lsems, rsems, lbsem, rbsem):
    wid = lax.axis_index("s") * _NC + lax.axis_index("c")
    base = wid * _BPW

    pltpu.sync_copy(left_hbm.at[pl.ds(base, _BPW)], lids)
    pltpu.sync_copy(right_hbm.at[pl.ds(base, _BPW)], rids)

    lane = lax.iota(jnp.int32, _L)
    cvec = lax.iota(jnp.int32, _L)

    def group(g, carry):
        gb = pl.multiple_of(g * _L, _L)
        vl = lids[pl.ds(gb, _L)]
        vr = rids[pl.ds(gb, _L)]
        lblks = (vl >> 7) << 7
        rblks = (vr >> 7) << 7
        llanes = vl & (_W - 1)
        rlanes = vr & (_W - 1)

        def fire(j):
            slot = j % _R
            cl = pltpu.async_copy(
                ltab_hbm.at[:, pl.ds(pl.multiple_of(lblks[j], _W), _W)],
                lblk.at[slot], lsems[slot])
            cr = pltpu.async_copy(
                rtab_hbm.at[:, pl.ds(pl.multiple_of(rblks[j], _W), _W)],
                rblk.at[slot], rsems[slot])
            return cl, cr

        bias_copies = []
        for j in range(_L):
            bias_copies.append(pltpu.async_copy(
                lbias_hbm.at[:, pl.ds(pl.multiple_of(lblks[j], _W), _W)],
                lbst.at[pl.ds(j, 1)], lbsem))
            bias_copies.append(pltpu.async_copy(
                rbias_hbm.at[:, pl.ds(pl.multiple_of(rblks[j], _W), _W)],
                rbst.at[pl.ds(j, 1)], rbsem))

        inflight = [fire(j) for j in range(_R)]
        for j in range(_L):
            slot = j % _R
            cl, cr = inflight[j]
            cl.wait()
            cr.wait()
            lj = llanes[j]
            rj = rlanes[j]
            for cc in range(0, _D, _L):
                lstage[j, pl.ds(cc, _L)] = plsc.load_gather(
                    lblk.at[slot], [cvec + cc, jnp.full((_L,), 0, jnp.int32) + lj])
                rstage[j, pl.ds(cc, _L)] = plsc.load_gather(
                    rblk.at[slot], [cvec + cc, jnp.full((_L,), 0, jnp.int32) + rj])
            if j + _R < _L:
                inflight.append(fire(j + _R))

        for c in bias_copies:
            c.wait()

        acc = plsc.load_gather(lbst, [lane, llanes]) + plsc.load_gather(
            rbst, [lane, rlanes])
        for c in range(_D):
            col = jnp.full((_L,), c, jnp.int32)
            acc = acc + plsc.load_gather(lstage, [lane, col]) * plsc.load_gather(
                rstage, [lane, col])
        outv[pl.ds(gb, _L)] = acc
        return carry

    lax.fori_loop(0, _NG, group, 0)

    pltpu.sync_copy(outv, out_hbm.at[pl.ds(base, _BPW)])


def kernel(left_id, right_id, l_emb, l_bias, r_emb, r_bias):
    return _glove_sc(
        left_id.astype(jnp.int32), right_id.astype(jnp.int32),
        l_emb.T, l_bias.T, r_emb.T, r_bias.T,
    )
